# baseline (speedup 1.0000x reference)
import functools

import jax
import jax.numpy as jnp
from jax import lax
from jax.experimental import pallas as pl
from jax.experimental.pallas import tpu as pltpu
from jax.experimental.pallas import tpu_sc as plsc

EPS = 1e-12


def kernel(input_ids, word_emb, pos_emb, gamma, beta):
    B, S = input_ids.shape
    V, H = word_emb.shape
    assert H == 64
    HP = 128
    NW = 32
    C = B // NW

    ids_t = input_ids.T
    wemb128 = jnp.pad(word_emb, ((0, 0), (0, HP - H)))

    mesh = plsc.VectorSubcoreMesh(core_axis_name="c", subcore_axis_name="s")

    @functools.partial(
        pl.kernel,
        out_type=jax.ShapeDtypeStruct((S, H, B), jnp.float32),
        mesh=mesh,
        scratch_types=[
            pltpu.VMEM((S, C), jnp.int32),
            pltpu.VMEM((2, C, HP), jnp.float32),
            pltpu.VMEM((2, H, C), jnp.float32),
            pltpu.VMEM((S, H), jnp.float32),
            pltpu.VMEM((H,), jnp.float32),
            pltpu.VMEM((H,), jnp.float32),
            pltpu.SemaphoreType.DMA,
            pltpu.SemaphoreType.DMA,
            pltpu.SemaphoreType.DMA,
            pltpu.SemaphoreType.DMA,
        ],
        compiler_params=pltpu.CompilerParams(needs_layout_passes=False),
    )
    def emb_ln(ids_hbm, wemb_hbm, pemb_hbm, g_hbm, b_hbm, out_hbm,
               ids_v, rows_v, outb_v, pos_v, g_v, b_v,
               gsem0, gsem1, osem0, osem1):
        cid = lax.axis_index("c")
        sid = lax.axis_index("s")
        wid = sid * 2 + cid
        b0 = wid * C
        gsem = [gsem0, gsem1]
        osem = [osem0, osem1]

        pltpu.sync_copy(ids_hbm.at[pl.ds(0, S), pl.ds(b0, C)], ids_v)
        pltpu.sync_copy(pemb_hbm.at[pl.ds(0, S)], pos_v)
        pltpu.sync_copy(g_hbm, g_v)
        pltpu.sync_copy(b_hbm, b_v)
        g = [g_v[pl.ds(16 * j, 16)] for j in range(4)]
        bt = [b_v[pl.ds(16 * j, 16)] for j in range(4)]
        iota = lax.iota(jnp.int32, 16)
        stages = (1, 2, 4, 8)
        conds = [(iota & s) == 0 for s in stages]
        perms = [iota ^ s for s in stages]

        def start_gather(t, b):
            pltpu.async_copy(wemb_hbm.at[ids_v.at[t]], rows_v.at[b], gsem[b])

        def wait_gather(t, b):
            pltpu.make_async_copy(
                wemb_hbm.at[ids_v.at[t]], rows_v.at[b], gsem[b]).wait()

        def drain_out(t, b):
            pltpu.make_async_copy(
                outb_v.at[b], out_hbm.at[t, :, pl.ds(b0, C)], osem[b]).wait()

        start_gather(0, 0)

        def pair_body(i2, carry):
            for b in range(2):
                t = i2 * 2 + b
                wait_gather(t, b)

                @pl.when(t + 1 < S)
                def _():
                    start_gather(t + 1, 1 - b)

                @pl.when(t >= 2)
                def _():
                    drain_out(t - 2, b)

                p = [pos_v[t, pl.ds(16 * j, 16)] for j in range(4)]
                @plsc.parallel_loop(0, C, unroll=8)
                def tok(k):
                    x = [rows_v[b, k, pl.ds(16 * j, 16)] + p[j]
                         for j in range(4)]
                    s = (x[0] + x[1]) + (x[2] + x[3])
                    q = ((x[0] * x[0] + x[1] * x[1])
                         + (x[2] * x[2] + x[3] * x[3]))
                    tot = jnp.take_along_axis(
                        plsc.cumsum(s), jnp.full((16,), 15, jnp.int32),
                        axis=0, mode="promise_in_bounds")
                    tot2 = jnp.take_along_axis(
                        plsc.cumsum(q), jnp.full((16,), 15, jnp.int32),
                        axis=0, mode="promise_in_bounds")
                    mean = tot * (1.0 / 64.0)
                    var = tot2 * (1.0 / 64.0) - mean * mean
                    v = jnp.maximum(var, 0.0) + EPS
                    iv = plsc.bitcast(v, jnp.int32)
                    y = plsc.bitcast(jnp.int32(0x5F3759DF) - (iv >> 1),
                                     jnp.float32)
                    h = 0.5 * v
                    y = y * (1.5 - h * y * y)
                    y = y * (1.5 - h * y * y)
                    for j in range(4):
                        rows_v[b, k, pl.ds(64 + 16 * j, 16)] = (
                            (x[j] - mean) * y * g[j] + bt[j])

                @plsc.parallel_loop(0, C // 16)
                def trans(kg):
                    k0 = kg * 16
                    for hg in range(4):
                        a = [rows_v[b, k0 + l, pl.ds(64 + 16 * hg, 16)]
                             for l in range(16)]
                        for si, s in enumerate(stages):
                            na = []
                            for i in range(16):
                                pr = jnp.take_along_axis(
                                    a[i ^ s], perms[si], axis=0,
                                    mode="promise_in_bounds")
                                if (i & s) == 0:
                                    na.append(jnp.where(conds[si], a[i], pr))
                                else:
                                    na.append(jnp.where(conds[si], pr, a[i]))
                            a = na
                        for m in range(16):
                            outb_v[b, 16 * hg + m, pl.ds(k0, 16)] = a[m]

                pltpu.async_copy(
                    outb_v.at[b], out_hbm.at[t, :, pl.ds(b0, C)], osem[b])
            return carry

        lax.fori_loop(0, S // 2, pair_body, 0)
        drain_out(S - 2, 0)
        drain_out(S - 1, 1)

    out3 = emb_ln(ids_t, wemb128, pos_emb, gamma, beta)
    return jnp.transpose(out3, (2, 0, 1))


# speedup vs baseline: 1.2406x; 1.2406x over previous
import functools

import jax
import jax.numpy as jnp
from jax import lax
from jax.experimental import pallas as pl
from jax.experimental.pallas import tpu as pltpu
from jax.experimental.pallas import tpu_sc as plsc

EPS = 1e-12


def kernel(input_ids, word_emb, pos_emb, gamma, beta):
    B, S = input_ids.shape
    V, H = word_emb.shape
    assert H == 64
    HP = 128
    NW = 32
    C = B // NW

    ids_t = input_ids.T
    wemb128 = jnp.pad(word_emb, ((0, 0), (0, HP - H)))

    mesh = plsc.VectorSubcoreMesh(core_axis_name="c", subcore_axis_name="s")

    @functools.partial(
        pl.kernel,
        out_type=jax.ShapeDtypeStruct((S, H, B), jnp.float32),
        mesh=mesh,
        scratch_types=[
            pltpu.VMEM((S, C), jnp.int32),
            pltpu.VMEM((2, C, HP), jnp.float32),
            pltpu.VMEM((2, H, C), jnp.float32),
            pltpu.VMEM((S, H), jnp.float32),
            pltpu.VMEM((H,), jnp.float32),
            pltpu.VMEM((H,), jnp.float32),
            pltpu.SemaphoreType.DMA,
            pltpu.SemaphoreType.DMA,
            pltpu.SemaphoreType.DMA,
            pltpu.SemaphoreType.DMA,
        ],
        compiler_params=pltpu.CompilerParams(needs_layout_passes=False),
    )
    def emb_ln(ids_hbm, wemb_hbm, pemb_hbm, g_hbm, b_hbm, out_hbm,
               ids_v, rows_v, outb_v, pos_v, g_v, b_v,
               gsem0, gsem1, osem0, osem1):
        cid = lax.axis_index("c")
        sid = lax.axis_index("s")
        wid = sid * 2 + cid
        b0 = wid * C
        gsem = [gsem0, gsem1]
        osem = [osem0, osem1]

        pltpu.sync_copy(ids_hbm.at[pl.ds(0, S), pl.ds(b0, C)], ids_v)
        pltpu.sync_copy(pemb_hbm.at[pl.ds(0, S)], pos_v)
        pltpu.sync_copy(g_hbm, g_v)
        pltpu.sync_copy(b_hbm, b_v)
        g = [g_v[pl.ds(16 * j, 16)] for j in range(4)]
        bt = [b_v[pl.ds(16 * j, 16)] for j in range(4)]
        iota = lax.iota(jnp.int32, 16)
        stages = (1, 2, 4, 8)
        conds = [(iota & s) == 0 for s in stages]
        perms = [iota ^ s for s in stages]

        def start_gather(t, b):
            pltpu.async_copy(wemb_hbm.at[ids_v.at[t]], rows_v.at[b], gsem[b])

        def wait_gather(t, b):
            pltpu.make_async_copy(
                wemb_hbm.at[ids_v.at[t]], rows_v.at[b], gsem[b]).wait()

        def drain_out(t, b):
            pltpu.make_async_copy(
                outb_v.at[b], out_hbm.at[t, :, pl.ds(b0, C)], osem[b]).wait()

        start_gather(0, 0)

        def pair_body(i2, carry):
            for b in range(2):
                t = i2 * 2 + b
                wait_gather(t, b)

                @pl.when(t + 1 < S)
                def _():
                    start_gather(t + 1, 1 - b)

                @pl.when(t >= 2)
                def _():
                    drain_out(t - 2, b)

                p = [pos_v[t, pl.ds(16 * j, 16)] for j in range(4)]
                @plsc.parallel_loop(0, C, unroll=4)
                def tok(k):
                    x = [rows_v[b, k, pl.ds(16 * j, 16)] + p[j]
                         for j in range(4)]
                    s = (x[0] + x[1]) + (x[2] + x[3])
                    q = ((x[0] * x[0] + x[1] * x[1])
                         + (x[2] * x[2] + x[3] * x[3]))
                    tot = jnp.take_along_axis(
                        plsc.cumsum(s), jnp.full((16,), 15, jnp.int32),
                        axis=0, mode="promise_in_bounds")
                    tot2 = jnp.take_along_axis(
                        plsc.cumsum(q), jnp.full((16,), 15, jnp.int32),
                        axis=0, mode="promise_in_bounds")
                    mean = tot * (1.0 / 64.0)
                    var = tot2 * (1.0 / 64.0) - mean * mean
                    v = jnp.maximum(var, 0.0) + EPS
                    iv = plsc.bitcast(v, jnp.int32)
                    y = plsc.bitcast(jnp.int32(0x5F3759DF) - (iv >> 1),
                                     jnp.float32)
                    h = 0.5 * v
                    y = y * (1.5 - h * y * y)
                    y = y * (1.5 - h * y * y)
                    for j in range(4):
                        rows_v[b, k, pl.ds(64 + 16 * j, 16)] = (
                            (x[j] - mean) * y * g[j] + bt[j])

                @plsc.parallel_loop(0, C // 16)
                def trans(kg):
                    k0 = kg * 16
                    for hg in range(4):
                        a = [rows_v[b, k0 + l, pl.ds(64 + 16 * hg, 16)]
                             for l in range(16)]
                        for si, s in enumerate(stages):
                            na = []
                            for i in range(16):
                                pr = jnp.take_along_axis(
                                    a[i ^ s], perms[si], axis=0,
                                    mode="promise_in_bounds")
                                if (i & s) == 0:
                                    na.append(jnp.where(conds[si], a[i], pr))
                                else:
                                    na.append(jnp.where(conds[si], pr, a[i]))
                            a = na
                        for m in range(16):
                            outb_v[b, 16 * hg + m, pl.ds(k0, 16)] = a[m]

                pltpu.async_copy(
                    outb_v.at[b], out_hbm.at[t, :, pl.ds(b0, C)], osem[b])
            return carry

        lax.fori_loop(0, S // 2, pair_body, 0)
        drain_out(S - 2, 0)
        drain_out(S - 1, 1)

    out3 = emb_ln(ids_t, wemb128, pos_emb, gamma, beta)
    return jnp.transpose(out3, (2, 0, 1))
